# baseline (speedup 1.0000x reference)
import functools

import jax
import jax.numpy as jnp
from jax import lax
from jax.experimental import pallas as pl
from jax.experimental.pallas import tpu as pltpu
from jax.experimental.pallas import tpu_sc as plsc

_NC = 2
_NS = 16
_L = 16
_NW = _NC * _NS
_CHUNK = 128


@functools.lru_cache(maxsize=None)
def _make_sc_kernel(batch: int, dim: int):
    b_w = batch // _NW
    n_chunks = b_w // _CHUNK
    n_groups = b_w // _L
    mesh = plsc.VectorSubcoreMesh(
        core_axis_name="c", subcore_axis_name="s",
        num_cores=_NC, num_subcores=_NS)

    @functools.partial(
        pl.kernel,
        out_type=jax.ShapeDtypeStruct((batch,), jnp.float32),
        mesh=mesh,
        compiler_params=pltpu.CompilerParams(
            needs_layout_passes=False, use_tc_tiling_on_sc=False),
        scratch_types=[
            pltpu.VMEM((b_w,), jnp.int32),
            pltpu.VMEM((b_w,), jnp.int32),
            pltpu.VMEM((b_w, dim), jnp.float32),
            pltpu.VMEM((b_w, dim), jnp.float32),
            pltpu.VMEM((b_w,), jnp.float32),
            pltpu.SemaphoreType.DMA,
        ],
    )
    def two_tower(uid_hbm, gid_hbm, ut_hbm, gt_hbm, out_hbm,
                  uidx, gidx, urows, grows, out_v, sem):
        wid = lax.axis_index("s") * _NC + lax.axis_index("c")
        base = wid * b_w
        pltpu.sync_copy(uid_hbm.at[pl.ds(base, b_w)], uidx)
        pltpu.sync_copy(gid_hbm.at[pl.ds(base, b_w)], gidx)

        copies = []
        for j in range(n_chunks):
            sl = pl.ds(j * _CHUNK, _CHUNK)
            copies.append(pltpu.async_copy(ut_hbm.at[uidx.at[sl]], urows.at[sl], sem))
            copies.append(pltpu.async_copy(gt_hbm.at[gidx.at[sl]], grows.at[sl], sem))
        for c in copies:
            c.wait()

        lane = lax.iota(jnp.int32, _L)

        def group(g, carry):
            rows = g * _L + lane
            acc = jnp.zeros((_L,), jnp.float32)
            for d in range(dim):
                cols = jnp.full((_L,), d, jnp.int32)
                u = plsc.load_gather(urows, [rows, cols])
                v = plsc.load_gather(grows, [rows, cols])
                acc = acc + u * v
            out_v[pl.ds(g * _L, _L)] = 1.0 / (1.0 + jnp.exp(-acc))
            return carry

        lax.fori_loop(0, n_groups, group, 0)
        pltpu.sync_copy(out_v, out_hbm.at[pl.ds(base, b_w)])

    return two_tower


def kernel(user_ids, game_ids, user_table, game_table):
    fn = _make_sc_kernel(user_ids.shape[0], user_table.shape[1])
    return fn(user_ids.astype(jnp.int32), game_ids.astype(jnp.int32),
              user_table, game_table)
